# baseline (speedup 1.0000x reference)
import functools

import jax
import jax.numpy as jnp
from jax import lax
from jax.experimental import pallas as pl
from jax.experimental.pallas import tpu as pltpu
from jax.experimental.pallas import tpu_sc as plsc


_G = 8


@functools.cache
def _make_gather(B, d_big, d_small, cpad, dpad, n_merch):
    info = plsc.get_sparse_core_info()
    NC, NS = info.num_cores, info.num_subcores
    NW = NC * NS
    assert B % NW == 0
    bpw = B // NW
    assert bpw % _G == 0
    ngroups = bpw // _G
    mesh = plsc.VectorSubcoreMesh(core_axis_name="c", subcore_axis_name="s")
    f32 = jnp.float32
    i32 = jnp.int32
    per_row = 128 // d_big
    nblocks = -(-n_merch // 128)
    m4_rows = nblocks * (128 // per_row)

    @functools.partial(
        pl.kernel,
        mesh=mesh,
        compiler_params=pltpu.CompilerParams(
            use_tc_tiling_on_sc=True, needs_layout_passes=False),
        out_type=[jax.ShapeDtypeStruct((B, 128), f32),
                  jax.ShapeDtypeStruct((m4_rows, 128), f32)],
        scratch_types=[
            pltpu.VMEM((bpw + 16,), i32),
            pltpu.VMEM((bpw,), i32),
            pltpu.VMEM((bpw + 16,), i32),
            pltpu.VMEM((bpw,), i32),
            pltpu.VMEM((bpw,), i32),
            pltpu.VMEM((bpw,), i32),
            pltpu.VMEM((_G, d_big, 128), f32),
            pltpu.VMEM((d_big, 128), f32),
            pltpu.VMEM((32, 128), f32),
            pltpu.VMEM((_G, 128), f32),
            pltpu.VMEM((d_small, cpad), f32),
            pltpu.VMEM((d_small, dpad), f32),
            pltpu.VMEM((_G, 128), f32),
            pltpu.SemaphoreType.DMA,
            pltpu.SemaphoreType.DMA,
        ],
    )
    def gather(userT, merchT, ctryT, devT, uq, ur, mq, mr, ci, di,
               xg_out, m4,
               uq_v, ur_v, mq_v, mr_v, ci_v, di_v,
               ustage, tin_v, tout_v, mrows_v, ctry_v, dev_v, xga_v,
               semu, semm):
        wid = lax.axis_index("s") * NC + lax.axis_index("c")
        base = wid * bpw
        sl = pl.ds(base, bpw)
        pltpu.sync_copy(uq.at[sl], uq_v.at[pl.ds(0, bpw)])
        pltpu.sync_copy(ur.at[sl], ur_v)
        pltpu.sync_copy(mq.at[sl], mq_v.at[pl.ds(0, bpw)])
        pltpu.sync_copy(mr.at[sl], mr_v)
        pltpu.sync_copy(ci.at[sl], ci_v)
        pltpu.sync_copy(di.at[sl], di_v)
        pltpu.sync_copy(ctryT, ctry_v)
        pltpu.sync_copy(devT, dev_v)

        iota16 = lax.iota(i32, 16)
        slot8 = iota16 % _G
        fpair = iota16 // _G

        sid = lax.axis_index("s")

        def tblock(bi, _):
            c = bi * NS + sid

            @pl.when(c < nblocks)
            def _():
                pltpu.sync_copy(
                    merchT.at[:, pl.ds(pl.multiple_of(c * 128, 128), 128)],
                    tin_v)

                def prow(qq, _):
                    qv = jnp.full((16,), qq, i32)
                    for j in range(per_row):
                        src = jnp.full((16,), per_row * qq + j, i32)
                        for h in range(d_big // 16):
                            v = plsc.load_gather(
                                tin_v, [h * 16 + iota16, src])
                            plsc.store_scatter(
                                tout_v, [qv, j * d_big + h * 16 + iota16], v)
                    return 0

                lax.fori_loop(0, 32, prow, 0)
                pltpu.sync_copy(tout_v, m4.at[pl.ds(c * 32, 32)])
            return 0

        lax.fori_loop(0, -(-nblocks // NS), tblock, 0)
        plsc.subcore_barrier()

        def fire_u(g):
            q_vec = uq_v[pl.ds(g * _G, 16)]
            for s in range(_G):
                col = pl.multiple_of(q_vec[s] * 128, 128)
                pltpu.async_copy(userT.at[:, pl.ds(col, 128)],
                                 ustage.at[s], semu)

        def drain_u():
            for s in range(_G):
                pltpu.make_async_copy(
                    userT.at[:, pl.ds(0, 128)], ustage.at[s], semu).wait()

        def extract_u(g):
            rv = plsc.load_gather(ur_v, [g * _G + slot8])
            for f in range(0, d_big, 2):
                v = plsc.load_gather(ustage, [slot8, fpair + f, rv])
                plsc.store_scatter(xga_v, [slot8, fpair + f], v)

        def extract_m(g):
            rv = plsc.load_gather(mr_v, [g * _G + slot8])
            for f in range(0, d_big, 2):
                v = plsc.load_gather(mrows_v, [slot8, rv + (fpair + f)])
                plsc.store_scatter(xga_v, [slot8, fpair + (d_big + f)], v)

        def extract_small(tab_v, idx_v, colbase, g):
            iv = plsc.load_gather(idx_v, [g * _G + slot8])
            for f in range(0, d_small, 2):
                v = plsc.load_gather(tab_v, [fpair + f, iv])
                plsc.store_scatter(
                    xga_v, [slot8, fpair + (colbase + f)], v)

        fire_u(0)

        def group(g, _):
            pltpu.async_copy(
                m4.at[mq_v.at[pl.ds(g * _G, _G)]], mrows_v, semm)
            drain_u()
            extract_u(g)

            @pl.when(g < ngroups - 1)
            def _():
                fire_u(g + 1)

            pltpu.make_async_copy(
                m4.at[pl.ds(0, _G)], mrows_v, semm).wait()
            extract_m(g)
            extract_small(ctry_v, ci_v, 2 * d_big, g)
            extract_small(dev_v, di_v, 2 * d_big + d_small, g)
            pltpu.sync_copy(xga_v, xg_out.at[pl.ds(base + g * _G, _G)])
            return 0

        lax.fori_loop(0, ngroups, group, 0)

    return gather


def _mlp_body(numT_r, xg_r, w1n_r, w1x_r, b1_r, w2_r, b2_r, w3_r, b3_r, out_r):
    f32 = jnp.float32
    dn = (((0,), (0,)), ((), ()))
    h = lax.dot_general(numT_r[...], w1n_r[...], dn, preferred_element_type=f32)
    h += jnp.dot(xg_r[:, :80], w1x_r[...], preferred_element_type=f32)
    h = jnp.maximum(h + b1_r[...], 0.0)
    h2 = jnp.dot(h, w2_r[...], preferred_element_type=f32)
    h2 = jnp.maximum(h2 + b2_r[...], 0.0)
    o = jnp.dot(h2, w3_r[...], preferred_element_type=f32)
    out_r[...] = jax.nn.sigmoid(o + b3_r[...])


@functools.cache
def _make_mlp(B, TB, n_num, H1, H2):
    grid = (B // TB,)

    def full(shape):
        return pl.BlockSpec(shape, lambda i: tuple(0 for _ in shape))

    return pl.pallas_call(
        _mlp_body,
        grid=grid,
        in_specs=[
            pl.BlockSpec((n_num, TB), lambda i: (0, i)),
            pl.BlockSpec((TB, 128), lambda i: (i, 0)),
            full((n_num, H1)), full((80, H1)), full((1, H1)),
            full((H1, H2)), full((1, H2)),
            full((H2, 1)), full((1, 1)),
        ],
        out_specs=pl.BlockSpec((TB, 1), lambda i: (i, 0)),
        out_shape=jax.ShapeDtypeStruct((B, 1), jnp.float32),
    )


def kernel(num, user, merchant, country, device,
           user_emb, merchant_emb, country_emb, device_emb,
           W1, b1, W2, b2, W3, b3):
    B, n_num = num.shape
    d_big = user_emb.shape[1]
    d_small = country_emb.shape[1]
    H1 = W1.shape[0]
    H2 = W2.shape[0]

    i32 = jnp.int32
    user = user.astype(i32)
    merchant = merchant.astype(i32)

    def pad128(n):
        return -(-n // 128) * 128

    cpad = pad128(country_emb.shape[0])
    dpad = pad128(device_emb.shape[0])
    ctryT = jnp.pad(country_emb, ((0, cpad - country_emb.shape[0]), (0, 0))).T
    devT = jnp.pad(device_emb, ((0, dpad - device_emb.shape[0]), (0, 0))).T

    per_row = 128 // d_big
    gather = _make_gather(B, d_big, d_small, cpad, dpad,
                          merchant_emb.shape[0])
    xg, _m4 = gather(user_emb.T, merchant_emb.T, ctryT, devT,
                     user // 128, user % 128,
                     merchant // per_row, (merchant % per_row) * d_big,
                     country.astype(i32), device.astype(i32))

    c0 = n_num
    w1n = W1[:, :c0].T
    w1x = W1[:, c0:].T
    mlp = _make_mlp(B, 2048, n_num, H1, H2)
    return mlp(num.T, xg,
               w1n, w1x, b1.reshape(1, H1),
               W2.T, b2.reshape(1, H2), W3.T, b3.reshape(1, 1))


# speedup vs baseline: 1.3756x; 1.3756x over previous
import functools

import jax
import jax.numpy as jnp
from jax import lax
from jax.experimental import pallas as pl
from jax.experimental.pallas import tpu as pltpu
from jax.experimental.pallas import tpu_sc as plsc


_G = 8


@functools.cache
def _make_gather(B, d_big, d_small, cpad, dpad):
    info = plsc.get_sparse_core_info()
    NC, NS = info.num_cores, info.num_subcores
    NW = NC * NS
    assert B % NW == 0
    bpw = B // NW
    assert bpw % _G == 0
    ngroups = bpw // _G
    mesh = plsc.VectorSubcoreMesh(core_axis_name="c", subcore_axis_name="s")
    f32 = jnp.float32
    i32 = jnp.int32

    @functools.partial(
        pl.kernel,
        mesh=mesh,
        compiler_params=pltpu.CompilerParams(
            use_tc_tiling_on_sc=True, needs_layout_passes=False),
        out_type=jax.ShapeDtypeStruct((B, 128), f32),
        scratch_types=[
            pltpu.VMEM((bpw + 16,), i32),
            pltpu.VMEM((bpw,), i32),
            pltpu.VMEM((bpw + 16,), i32),
            pltpu.VMEM((bpw,), i32),
            pltpu.VMEM((bpw,), i32),
            pltpu.VMEM((bpw,), i32),
            pltpu.VMEM((2, _G, d_big, 128), f32),
            pltpu.VMEM((_G, d_big, 128), f32),
            pltpu.VMEM((d_small, cpad), f32),
            pltpu.VMEM((d_small, dpad), f32),
            pltpu.VMEM((_G, 128), f32),
            pltpu.SemaphoreType.DMA,
            pltpu.SemaphoreType.DMA,
        ],
    )
    def gather(userT, merchT, ctryT, devT, uq, ur, mq, mr, ci, di,
               xg_out,
               uq_v, ur_v, mq_v, mr_v, ci_v, di_v,
               ustage, mstage, ctry_v, dev_v, xga_v, semu, semm):
        wid = lax.axis_index("s") * NC + lax.axis_index("c")
        base = wid * bpw
        sl = pl.ds(base, bpw)
        pltpu.sync_copy(uq.at[sl], uq_v.at[pl.ds(0, bpw)])
        pltpu.sync_copy(ur.at[sl], ur_v)
        pltpu.sync_copy(mq.at[sl], mq_v.at[pl.ds(0, bpw)])
        pltpu.sync_copy(mr.at[sl], mr_v)
        pltpu.sync_copy(ci.at[sl], ci_v)
        pltpu.sync_copy(di.at[sl], di_v)
        pltpu.sync_copy(ctryT, ctry_v)
        pltpu.sync_copy(devT, dev_v)

        iota16 = lax.iota(i32, 16)
        slot8 = iota16 % _G
        fpair = iota16 // _G

        def fire_u(g):
            b = g % 2
            q_vec = uq_v[pl.ds(g * _G, 16)]
            for s in range(_G):
                col = pl.multiple_of(q_vec[s] * 128, 128)
                pltpu.async_copy(userT.at[:, pl.ds(col, 128)],
                                 ustage.at[b, s], semu)

        def fire_m(g):
            q_vec = mq_v[pl.ds(g * _G, 16)]
            for s in range(_G):
                col = pl.multiple_of(q_vec[s] * 128, 128)
                pltpu.async_copy(merchT.at[:, pl.ds(col, 128)],
                                 mstage.at[s], semm)

        def drain(tabT, sem):
            for s in range(_G):
                pltpu.make_async_copy(
                    tabT.at[:, pl.ds(0, 128)], mstage.at[s], sem).wait()

        def extract_u(g):
            bvec = jnp.full((16,), g % 2, i32)
            rv = plsc.load_gather(ur_v, [g * _G + slot8])
            for f in range(0, d_big, 2):
                v = plsc.load_gather(ustage, [bvec, slot8, fpair + f, rv])
                plsc.store_scatter(xga_v, [slot8, fpair + f], v)

        def extract_m(g):
            rv = plsc.load_gather(mr_v, [g * _G + slot8])
            for f in range(0, d_big, 2):
                v = plsc.load_gather(mstage, [slot8, fpair + f, rv])
                plsc.store_scatter(xga_v, [slot8, fpair + (d_big + f)], v)

        def extract_small(tab_v, idx_v, colbase, g):
            iv = plsc.load_gather(idx_v, [g * _G + slot8])
            for f in range(0, d_small, 2):
                v = plsc.load_gather(tab_v, [fpair + f, iv])
                plsc.store_scatter(
                    xga_v, [slot8, fpair + (colbase + f)], v)

        fire_u(0)

        def group(g, _):
            fire_m(g)

            @pl.when(g < ngroups - 1)
            def _():
                fire_u(g + 1)

            drain(userT, semu)
            extract_u(g)
            drain(merchT, semm)
            extract_m(g)
            extract_small(ctry_v, ci_v, 2 * d_big, g)
            extract_small(dev_v, di_v, 2 * d_big + d_small, g)
            pltpu.sync_copy(xga_v, xg_out.at[pl.ds(base + g * _G, _G)])
            return 0

        lax.fori_loop(0, ngroups, group, 0)

    return gather


def _mlp_body(numT_r, xg_r, w1n_r, w1x_r, b1_r, w2_r, b2_r, w3_r, b3_r, out_r):
    f32 = jnp.float32
    dn = (((0,), (0,)), ((), ()))
    h = lax.dot_general(numT_r[...], w1n_r[...], dn, preferred_element_type=f32)
    h += jnp.dot(xg_r[:, :80], w1x_r[...], preferred_element_type=f32)
    h = jnp.maximum(h + b1_r[...], 0.0)
    h2 = jnp.dot(h, w2_r[...], preferred_element_type=f32)
    h2 = jnp.maximum(h2 + b2_r[...], 0.0)
    o = jnp.dot(h2, w3_r[...], preferred_element_type=f32)
    out_r[...] = jax.nn.sigmoid(o + b3_r[...])


@functools.cache
def _make_mlp(B, TB, n_num, H1, H2):
    grid = (B // TB,)

    def full(shape):
        return pl.BlockSpec(shape, lambda i: tuple(0 for _ in shape))

    return pl.pallas_call(
        _mlp_body,
        grid=grid,
        in_specs=[
            pl.BlockSpec((n_num, TB), lambda i: (0, i)),
            pl.BlockSpec((TB, 128), lambda i: (i, 0)),
            full((n_num, H1)), full((80, H1)), full((1, H1)),
            full((H1, H2)), full((1, H2)),
            full((H2, 1)), full((1, 1)),
        ],
        out_specs=pl.BlockSpec((TB, 1), lambda i: (i, 0)),
        out_shape=jax.ShapeDtypeStruct((B, 1), jnp.float32),
    )


def kernel(num, user, merchant, country, device,
           user_emb, merchant_emb, country_emb, device_emb,
           W1, b1, W2, b2, W3, b3):
    B, n_num = num.shape
    d_big = user_emb.shape[1]
    d_small = country_emb.shape[1]
    H1 = W1.shape[0]
    H2 = W2.shape[0]

    i32 = jnp.int32
    user = user.astype(i32)
    merchant = merchant.astype(i32)

    def pad128(n):
        return -(-n // 128) * 128

    cpad = pad128(country_emb.shape[0])
    dpad = pad128(device_emb.shape[0])
    ctryT = jnp.pad(country_emb, ((0, cpad - country_emb.shape[0]), (0, 0))).T
    devT = jnp.pad(device_emb, ((0, dpad - device_emb.shape[0]), (0, 0))).T

    gather = _make_gather(B, d_big, d_small, cpad, dpad)
    xg = gather(user_emb.T, merchant_emb.T, ctryT, devT,
                user // 128, user % 128, merchant // 128, merchant % 128,
                country.astype(i32), device.astype(i32))

    c0 = n_num
    w1n = W1[:, :c0].T
    w1x = W1[:, c0:].T
    mlp = _make_mlp(B, 2048, n_num, H1, H2)
    return mlp(num.T, xg,
               w1n, w1x, b1.reshape(1, H1),
               W2.T, b2.reshape(1, H2), W3.T, b3.reshape(1, 1))


# speedup vs baseline: 1.4044x; 1.0209x over previous
import functools

import jax
import jax.numpy as jnp
from jax import lax
from jax.experimental import pallas as pl
from jax.experimental.pallas import tpu as pltpu
from jax.experimental.pallas import tpu_sc as plsc


_G = 8


@functools.cache
def _make_gather(B, d_big, d_small, cpad, dpad):
    info = plsc.get_sparse_core_info()
    NC, NS = info.num_cores, info.num_subcores
    NW = NC * NS
    assert B % NW == 0
    bpw = B // NW
    assert bpw % _G == 0
    ngroups = bpw // _G
    mesh = plsc.VectorSubcoreMesh(core_axis_name="c", subcore_axis_name="s")
    f32 = jnp.float32
    i32 = jnp.int32

    @functools.partial(
        pl.kernel,
        mesh=mesh,
        compiler_params=pltpu.CompilerParams(
            use_tc_tiling_on_sc=True, needs_layout_passes=False),
        out_type=jax.ShapeDtypeStruct((B, 128), f32),
        scratch_types=[
            pltpu.VMEM((bpw + 16,), i32),
            pltpu.VMEM((bpw,), i32),
            pltpu.VMEM((bpw + 16,), i32),
            pltpu.VMEM((bpw,), i32),
            pltpu.VMEM((bpw,), i32),
            pltpu.VMEM((bpw,), i32),
            pltpu.VMEM((_G, d_big, 128), f32),
            pltpu.VMEM((_G, d_big, 128), f32),
            pltpu.VMEM((d_small, cpad), f32),
            pltpu.VMEM((d_small, dpad), f32),
            pltpu.VMEM((_G, 128), f32),
            pltpu.SemaphoreType.DMA,
            pltpu.SemaphoreType.DMA,
        ],
    )
    def gather(userT, merchT, ctryT, devT, uq, ur, mq, mr, ci, di,
               xg_out,
               uq_v, ur_v, mq_v, mr_v, ci_v, di_v,
               ustage, mstage, ctry_v, dev_v, xga_v, semu, semm):
        wid = lax.axis_index("s") * NC + lax.axis_index("c")
        base = wid * bpw
        sl = pl.ds(base, bpw)
        pltpu.sync_copy(uq.at[sl], uq_v.at[pl.ds(0, bpw)])
        pltpu.sync_copy(ur.at[sl], ur_v)
        pltpu.sync_copy(mq.at[sl], mq_v.at[pl.ds(0, bpw)])
        pltpu.sync_copy(mr.at[sl], mr_v)
        pltpu.sync_copy(ci.at[sl], ci_v)
        pltpu.sync_copy(di.at[sl], di_v)
        pltpu.sync_copy(ctryT, ctry_v)
        pltpu.sync_copy(devT, dev_v)

        iota16 = lax.iota(i32, 16)
        slot8 = iota16 % _G
        fpair = iota16 // _G

        def fire_u(g):
            q_vec = uq_v[pl.ds(g * _G, 16)]
            for s in range(_G):
                col = pl.multiple_of(q_vec[s] * 128, 128)
                pltpu.async_copy(userT.at[:, pl.ds(col, 128)],
                                 ustage.at[s], semu)

        def fire_m(g):
            q_vec = mq_v[pl.ds(g * _G, 16)]
            for s in range(_G):
                col = pl.multiple_of(q_vec[s] * 128, 128)
                pltpu.async_copy(merchT.at[:, pl.ds(col, 128)],
                                 mstage.at[s], semm)

        def drain(tabT, sem):
            for s in range(_G):
                pltpu.make_async_copy(
                    tabT.at[:, pl.ds(0, 128)], mstage.at[s], sem).wait()

        def extract_u(g):
            rv = plsc.load_gather(ur_v, [g * _G + slot8])
            for f in range(0, d_big, 2):
                v = plsc.load_gather(ustage, [slot8, fpair + f, rv])
                plsc.store_scatter(xga_v, [slot8, fpair + f], v)

        def extract_m(g):
            rv = plsc.load_gather(mr_v, [g * _G + slot8])
            for f in range(0, d_big, 2):
                v = plsc.load_gather(mstage, [slot8, fpair + f, rv])
                plsc.store_scatter(xga_v, [slot8, fpair + (d_big + f)], v)

        def extract_small(tab_v, idx_v, colbase, g):
            iv = plsc.load_gather(idx_v, [g * _G + slot8])
            for f in range(0, d_small, 2):
                v = plsc.load_gather(tab_v, [fpair + f, iv])
                plsc.store_scatter(
                    xga_v, [slot8, fpair + (colbase + f)], v)

        fire_u(0)

        def group(g, _):
            fire_m(g)
            drain(userT, semu)
            extract_u(g)

            @pl.when(g < ngroups - 1)
            def _():
                fire_u(g + 1)

            drain(merchT, semm)
            extract_m(g)
            extract_small(ctry_v, ci_v, 2 * d_big, g)
            extract_small(dev_v, di_v, 2 * d_big + d_small, g)
            pltpu.sync_copy(xga_v, xg_out.at[pl.ds(base + g * _G, _G)])
            return 0

        lax.fori_loop(0, ngroups, group, 0)

    return gather


def _mlp_body(numT_r, xg_r, w1n_r, w1x_r, b1_r, w2_r, b2_r, w3_r, b3_r, out_r):
    f32 = jnp.float32
    dn = (((0,), (0,)), ((), ()))
    h = lax.dot_general(numT_r[...], w1n_r[...], dn, preferred_element_type=f32)
    h += jnp.dot(xg_r[:, :80], w1x_r[...], preferred_element_type=f32)
    h = jnp.maximum(h + b1_r[...], 0.0)
    h2 = jnp.dot(h, w2_r[...], preferred_element_type=f32)
    h2 = jnp.maximum(h2 + b2_r[...], 0.0)
    o = jnp.dot(h2, w3_r[...], preferred_element_type=f32)
    out_r[...] = jax.nn.sigmoid(o + b3_r[...])[:, 0]


@functools.cache
def _make_mlp(B, TB, n_num, H1, H2):
    grid = (B // TB,)

    def full(shape):
        return pl.BlockSpec(shape, lambda i: tuple(0 for _ in shape))

    return pl.pallas_call(
        _mlp_body,
        grid=grid,
        in_specs=[
            pl.BlockSpec((n_num, TB), lambda i: (0, i)),
            pl.BlockSpec((TB, 128), lambda i: (i, 0)),
            full((n_num, H1)), full((80, H1)), full((1, H1)),
            full((H1, H2)), full((1, H2)),
            full((H2, 1)), full((1, 1)),
        ],
        out_specs=pl.BlockSpec((TB,), lambda i: (i,)),
        out_shape=jax.ShapeDtypeStruct((B,), jnp.float32),
    )


def kernel(num, user, merchant, country, device,
           user_emb, merchant_emb, country_emb, device_emb,
           W1, b1, W2, b2, W3, b3):
    B, n_num = num.shape
    d_big = user_emb.shape[1]
    d_small = country_emb.shape[1]
    H1 = W1.shape[0]
    H2 = W2.shape[0]

    i32 = jnp.int32
    user = user.astype(i32)
    merchant = merchant.astype(i32)

    def pad128(n):
        return -(-n // 128) * 128

    cpad = pad128(country_emb.shape[0])
    dpad = pad128(device_emb.shape[0])
    ctryT = jnp.pad(country_emb, ((0, cpad - country_emb.shape[0]), (0, 0))).T
    devT = jnp.pad(device_emb, ((0, dpad - device_emb.shape[0]), (0, 0))).T

    gather = _make_gather(B, d_big, d_small, cpad, dpad)
    xg = gather(user_emb.T, merchant_emb.T, ctryT, devT,
                user // 128, user % 128, merchant // 128, merchant % 128,
                country.astype(i32), device.astype(i32))

    c0 = n_num
    w1n = W1[:, :c0].T
    w1x = W1[:, c0:].T
    mlp = _make_mlp(B, 4096, n_num, H1, H2)
    out = mlp(num.T, xg,
              w1n, w1x, b1.reshape(1, H1),
              W2.T, b2.reshape(1, H2), W3.T, b3.reshape(1, 1))
    return out.reshape(B, 1)


# speedup vs baseline: 1.4351x; 1.0219x over previous
import functools

import jax
import jax.numpy as jnp
from jax import lax
from jax.experimental import pallas as pl
from jax.experimental.pallas import tpu as pltpu
from jax.experimental.pallas import tpu_sc as plsc


_G = 8


@functools.cache
def _make_gather(B, d_big, d_small, cpad, dpad):
    info = plsc.get_sparse_core_info()
    NC, NS = info.num_cores, info.num_subcores
    NW = NC * NS
    assert B % NW == 0
    bpw = B // NW
    assert bpw % _G == 0
    ngroups = bpw // _G
    mesh = plsc.VectorSubcoreMesh(core_axis_name="c", subcore_axis_name="s")
    f32 = jnp.float32
    i32 = jnp.int32

    @functools.partial(
        pl.kernel,
        mesh=mesh,
        compiler_params=pltpu.CompilerParams(
            use_tc_tiling_on_sc=True, needs_layout_passes=False),
        out_type=jax.ShapeDtypeStruct((B, 128), f32),
        scratch_types=[
            pltpu.VMEM((bpw + 16,), i32),
            pltpu.VMEM((bpw,), i32),
            pltpu.VMEM((bpw + 16,), i32),
            pltpu.VMEM((bpw,), i32),
            pltpu.VMEM((bpw,), i32),
            pltpu.VMEM((bpw,), i32),
            pltpu.VMEM((_G, d_big, 128), f32),
            pltpu.VMEM((_G, d_big, 128), f32),
            pltpu.VMEM((d_small, cpad), f32),
            pltpu.VMEM((d_small, dpad), f32),
            pltpu.VMEM((2, _G, 128), f32),
            pltpu.SemaphoreType.DMA,
            pltpu.SemaphoreType.DMA,
            pltpu.SemaphoreType.DMA,
        ],
    )
    def gather(userT, merchT, ctryT, devT, uq, ur, mq, mr, ci, di,
               xg_out,
               uq_v, ur_v, mq_v, mr_v, ci_v, di_v,
               ustage, mstage, ctry_v, dev_v, xga_v, semu, semm, semo):
        wid = lax.axis_index("s") * NC + lax.axis_index("c")
        base = wid * bpw
        sl = pl.ds(base, bpw)
        pltpu.sync_copy(uq.at[sl], uq_v.at[pl.ds(0, bpw)])
        pltpu.sync_copy(ur.at[sl], ur_v)
        pltpu.sync_copy(mq.at[sl], mq_v.at[pl.ds(0, bpw)])
        pltpu.sync_copy(mr.at[sl], mr_v)
        pltpu.sync_copy(ci.at[sl], ci_v)
        pltpu.sync_copy(di.at[sl], di_v)
        pltpu.sync_copy(ctryT, ctry_v)
        pltpu.sync_copy(devT, dev_v)

        iota16 = lax.iota(i32, 16)
        slot8 = iota16 % _G
        fpair = iota16 // _G

        def fire_u(g):
            q_vec = uq_v[pl.ds(g * _G, 16)]
            for s in range(_G):
                col = pl.multiple_of(q_vec[s] * 128, 128)
                pltpu.async_copy(userT.at[:, pl.ds(col, 128)],
                                 ustage.at[s], semu)

        def fire_m(g):
            q_vec = mq_v[pl.ds(g * _G, 16)]
            for s in range(_G):
                col = pl.multiple_of(q_vec[s] * 128, 128)
                pltpu.async_copy(merchT.at[:, pl.ds(col, 128)],
                                 mstage.at[s], semm)

        def drain(tabT, sem):
            for s in range(_G):
                pltpu.make_async_copy(
                    tabT.at[:, pl.ds(0, 128)], mstage.at[s], sem).wait()

        def extract_u(g):
            bvec = jnp.full((16,), g % 2, i32)
            rv = plsc.load_gather(ur_v, [g * _G + slot8])
            for f in range(0, d_big, 2):
                v = plsc.load_gather(ustage, [slot8, fpair + f, rv])
                plsc.store_scatter(xga_v, [bvec, slot8, fpair + f], v)

        def extract_m(g):
            bvec = jnp.full((16,), g % 2, i32)
            rv = plsc.load_gather(mr_v, [g * _G + slot8])
            for f in range(0, d_big, 2):
                v = plsc.load_gather(mstage, [slot8, fpair + f, rv])
                plsc.store_scatter(xga_v, [bvec, slot8, fpair + (d_big + f)], v)

        def extract_small(tab_v, idx_v, colbase, g):
            bvec = jnp.full((16,), g % 2, i32)
            iv = plsc.load_gather(idx_v, [g * _G + slot8])
            for f in range(0, d_small, 2):
                v = plsc.load_gather(tab_v, [fpair + f, iv])
                plsc.store_scatter(
                    xga_v, [bvec, slot8, fpair + (colbase + f)], v)

        fire_u(0)

        def group(g, _):
            fire_m(g)
            drain(userT, semu)
            extract_u(g)

            @pl.when(g < ngroups - 1)
            def _():
                fire_u(g + 1)

            extract_small(ctry_v, ci_v, 2 * d_big, g)
            extract_small(dev_v, di_v, 2 * d_big + d_small, g)

            @pl.when(g > 0)
            def _():
                pltpu.make_async_copy(
                    xga_v.at[0], xg_out.at[pl.ds(base, _G)], semo).wait()

            drain(merchT, semm)
            extract_m(g)
            pltpu.async_copy(
                xga_v.at[g % 2], xg_out.at[pl.ds(base + g * _G, _G)], semo)
            return 0

        lax.fori_loop(0, ngroups, group, 0)
        pltpu.make_async_copy(
            xga_v.at[0], xg_out.at[pl.ds(base, _G)], semo).wait()

    return gather


def _mlp_body(numT_r, xg_r, w1n_r, w1x_r, b1_r, w2_r, b2_r, w3_r, b3_r, out_r):
    f32 = jnp.float32
    dn = (((0,), (0,)), ((), ()))
    h = lax.dot_general(numT_r[...], w1n_r[...], dn, preferred_element_type=f32)
    h += jnp.dot(xg_r[:, :80], w1x_r[...], preferred_element_type=f32)
    h = jnp.maximum(h + b1_r[...], 0.0)
    h2 = jnp.dot(h, w2_r[...], preferred_element_type=f32)
    h2 = jnp.maximum(h2 + b2_r[...], 0.0)
    o = jnp.dot(h2, w3_r[...], preferred_element_type=f32)
    out_r[...] = jax.nn.sigmoid(o + b3_r[...])[:, 0]


@functools.cache
def _make_mlp(B, TB, n_num, H1, H2):
    grid = (B // TB,)

    def full(shape):
        return pl.BlockSpec(shape, lambda i: tuple(0 for _ in shape))

    return pl.pallas_call(
        _mlp_body,
        grid=grid,
        in_specs=[
            pl.BlockSpec((n_num, TB), lambda i: (0, i)),
            pl.BlockSpec((TB, 128), lambda i: (i, 0)),
            full((n_num, H1)), full((80, H1)), full((1, H1)),
            full((H1, H2)), full((1, H2)),
            full((H2, 1)), full((1, 1)),
        ],
        out_specs=pl.BlockSpec((TB,), lambda i: (i,)),
        out_shape=jax.ShapeDtypeStruct((B,), jnp.float32),
    )


def kernel(num, user, merchant, country, device,
           user_emb, merchant_emb, country_emb, device_emb,
           W1, b1, W2, b2, W3, b3):
    B, n_num = num.shape
    d_big = user_emb.shape[1]
    d_small = country_emb.shape[1]
    H1 = W1.shape[0]
    H2 = W2.shape[0]

    i32 = jnp.int32
    user = user.astype(i32)
    merchant = merchant.astype(i32)

    def pad128(n):
        return -(-n // 128) * 128

    cpad = pad128(country_emb.shape[0])
    dpad = pad128(device_emb.shape[0])
    ctryT = jnp.pad(country_emb, ((0, cpad - country_emb.shape[0]), (0, 0))).T
    devT = jnp.pad(device_emb, ((0, dpad - device_emb.shape[0]), (0, 0))).T

    gather = _make_gather(B, d_big, d_small, cpad, dpad)
    xg = gather(user_emb.T, merchant_emb.T, ctryT, devT,
                user // 128, user % 128, merchant // 128, merchant % 128,
                country.astype(i32), device.astype(i32))

    c0 = n_num
    w1n = W1[:, :c0].T
    w1x = W1[:, c0:].T
    mlp = _make_mlp(B, 4096, n_num, H1, H2)
    out = mlp(num.T, xg,
              w1n, w1x, b1.reshape(1, H1),
              W2.T, b2.reshape(1, H2), W3.T, b3.reshape(1, 1))
    return out.reshape(B, 1)
